# baseline (device time: 123327 ns/iter reference)
import jax
import jax.numpy as jnp
from jax import lax
from jax.experimental import pallas as pl
from jax.experimental.pallas import tpu as pltpu

N_DEV = 4
SQ = 1024
SKV = 1024
HQ = 32
HQ_LOC = 8
DH = 128
BLK = 64
N_CHUNK = 4
CS = SQ // N_CHUNK
SCALE = 0.08838834764831843
NEG = -1e9
MESH = pl.DeviceIdType.MESH


def kernel(x, Wq, K_ext, V_ext, Wo):
    B, Sq, D = x.shape

    def body(x_ref, wq_ref, k_ext_ref, v_ext_ref, wo_ref, out_ref,
             stage, ksend16, vsend16, k_buf, v_buf, q_hm,
             rs_send_buf, rs_buf, ag_src, ag_buf, relay_buf,
             kv_send_sems, k_recv_sems, v_recv_sems,
             rs_send_sems, rs_recv_sems, ag_send_sems, ag_recv_sems,
             stage_sems, relay_recv_sems, relay_send_sems):
        my = lax.axis_index("i")

        def stage_dma(ext_ref, c, slot):
            return pltpu.make_async_copy(
                ext_ref.at[0, pl.ds(CS * c, CS)], stage.at[slot],
                stage_sems.at[slot])

        @pl.when(my == 0)
        def _():
            stage_dma(k_ext_ref, 0, 0).start()
            stage_dma(v_ext_ref, 0, 1).start()

        bsem = pltpu.get_barrier_semaphore()
        for k in range(1, N_DEV):
            peer = (my + k) % N_DEV
            pl.semaphore_signal(bsem, inc=1, device_id=(peer,),
                                device_id_type=MESH)
        pl.semaphore_wait(bsem, N_DEV - 1)

        @pl.when(my == 0)
        def _():
            for c in range(N_CHUNK):
                for slot, (ext_ref, own_buf, send16, rsems) in enumerate([
                        (k_ext_ref, k_buf, ksend16, k_recv_sems),
                        (v_ext_ref, v_buf, vsend16, v_recv_sems)]):
                    stage_dma(ext_ref, c, slot).wait()
                    val16 = stage[slot].astype(jnp.bfloat16)
                    for h in range(HQ_LOC):
                        own_buf[h, pl.ds(CS * c, CS), :] = val16[:, h, :]
                    for j in range(1, N_DEV):
                        for h in range(HQ_LOC):
                            send16[j - 1, h, pl.ds(CS * c, CS), :] = (
                                val16[:, HQ_LOC * j + h, :])
                    if c + 1 < N_CHUNK:
                        stage_dma(ext_ref, c + 1, slot).start()
                    for j in range(1, N_DEV):
                        src = send16.at[j - 1, :, pl.ds(CS * c, CS), :]
                        if j == 2:
                            r = pltpu.make_async_remote_copy(
                                src_ref=src,
                                dst_ref=relay_buf.at[c],
                                send_sem=kv_send_sems.at[c * 6 + 2 + slot],
                                recv_sem=relay_recv_sems.at[c + N_CHUNK * slot],
                                device_id=(3 - 2 * slot,),
                                device_id_type=MESH,
                            )
                        else:
                            r = pltpu.make_async_remote_copy(
                                src_ref=src,
                                dst_ref=own_buf.at[:, pl.ds(CS * c, CS), :],
                                send_sem=kv_send_sems.at[c * 6 + (j - 1) * 2
                                                         + slot],
                                recv_sem=rsems.at[c],
                                device_id=(j,), device_id_type=MESH,
                            )
                        r.start()

        x16 = x_ref[0, :, :].astype(jnp.bfloat16)
        wq16 = wq_ref[:, :].astype(jnp.bfloat16)
        q2d = jnp.dot(x16, wq16, preferred_element_type=jnp.float32) * SCALE
        for h in range(HQ_LOC):
            q_hm[h] = q2d[:, h * DH:(h + 1) * DH].astype(jnp.bfloat16)
        wo16 = wo_ref[:, :].astype(jnp.bfloat16)

        keep_diag = (lax.broadcasted_iota(jnp.int32, (CS, CS), 1) // BLK
                     <= lax.broadcasted_iota(jnp.int32, (CS, CS), 0) // BLK)

        def qk_tile(s, c, h):
            qh = q_hm[h, pl.ds(CS * s, CS), :]
            kh = k_buf[h, pl.ds(CS * c, CS), :]
            sc = lax.dot_general(qh, kh, (((1,), (1,)), ((), ())),
                                 preferred_element_type=jnp.float32)
            if s == c:
                sc = jnp.where(keep_diag, sc, NEG)
            w = jnp.exp(sc)
            vh = v_buf[h, pl.ds(CS * c, CS), :]
            pv = jnp.dot(w.astype(jnp.bfloat16), vh,
                         preferred_element_type=jnp.float32)
            return pv, jnp.sum(w, axis=1, keepdims=True)

        ctx_acc = [[None] * HQ_LOC for _ in range(N_CHUNK)]
        den_acc = [[None] * HQ_LOC for _ in range(N_CHUNK)]

        def add_tile(s, c, h):
            pv, den = qk_tile(s, c, h)
            if ctx_acc[s][h] is None:
                ctx_acc[s][h], den_acc[s][h] = pv, den
            else:
                ctx_acc[s][h] = ctx_acc[s][h] + pv
                den_acc[s][h] = den_acc[s][h] + den

        for c in range(N_CHUNK):
            @pl.when(my != 0)
            def _(c=c):
                kd = pltpu.make_async_remote_copy(
                    src_ref=k_buf.at[:, pl.ds(CS * c, CS), :],
                    dst_ref=k_buf.at[:, pl.ds(CS * c, CS), :],
                    send_sem=kv_send_sems.at[0], recv_sem=k_recv_sems.at[c],
                    device_id=(0,), device_id_type=MESH,
                )
                kd.wait_recv()
                vd = pltpu.make_async_remote_copy(
                    src_ref=v_buf.at[:, pl.ds(CS * c, CS), :],
                    dst_ref=v_buf.at[:, pl.ds(CS * c, CS), :],
                    send_sem=kv_send_sems.at[1], recv_sem=v_recv_sems.at[c],
                    device_id=(0,), device_id_type=MESH,
                )
                vd.wait_recv()

            for relayer, slot, dbuf, dsems in ((3, 0, k_buf, k_recv_sems),
                                               (1, 1, v_buf, v_recv_sems)):
                @pl.when(my == relayer)
                def _(c=c, slot=slot, dbuf=dbuf, dsems=dsems):
                    rin = pltpu.make_async_remote_copy(
                        src_ref=relay_buf.at[c], dst_ref=relay_buf.at[c],
                        send_sem=kv_send_sems.at[0],
                        recv_sem=relay_recv_sems.at[c + N_CHUNK * slot],
                        device_id=(0,), device_id_type=MESH,
                    )
                    rin.wait_recv()
                    fwd = pltpu.make_async_remote_copy(
                        src_ref=relay_buf.at[c],
                        dst_ref=dbuf.at[:, pl.ds(CS * c, CS), :],
                        send_sem=relay_send_sems.at[c],
                        recv_sem=dsems.at[c],
                        device_id=(2,), device_id_type=MESH,
                    )
                    fwd.start()

            for h in range(HQ_LOC):
                add_tile(c, c, h)
            ctx16 = jnp.concatenate(
                [ctx_acc[c][h] / den_acc[c][h] for h in range(HQ_LOC)],
                axis=1).astype(jnp.bfloat16)
            partial_c = jnp.dot(ctx16, wo16,
                                preferred_element_type=jnp.float32)

            @pl.when(my == c)
            def _(c=c, partial_c=partial_c):
                out_ref[0, CS * c:CS * (c + 1), :] = partial_c

            @pl.when(my != c)
            def _(c=c, partial_c=partial_c):
                rs_send_buf[c] = partial_c.astype(jnp.bfloat16)
                rs = pltpu.make_async_remote_copy(
                    src_ref=rs_send_buf.at[c],
                    dst_ref=rs_buf.at[my],
                    send_sem=rs_send_sems.at[c],
                    recv_sem=rs_recv_sems.at[my],
                    device_id=(c,), device_id_type=MESH,
                )
                rs.start()

            for s in range(c + 1, N_CHUNK):
                for h in range(HQ_LOC):
                    add_tile(s, c, h)

        total = out_ref[0, pl.ds(my * CS, CS), :]
        for k in range(1, N_DEV):
            peer = (my + k) % N_DEV
            rd = pltpu.make_async_remote_copy(
                src_ref=rs_send_buf.at[0],
                dst_ref=rs_buf.at[peer],
                send_sem=rs_send_sems.at[0],
                recv_sem=rs_recv_sems.at[peer],
                device_id=(peer,), device_id_type=MESH,
            )
            rd.wait_recv()
            total = total + rs_buf[peer].astype(jnp.float32)

        ag_src[...] = total.astype(jnp.bfloat16)
        out_ref[0, pl.ds(my * CS, CS), :] = total
        for k in range(1, N_DEV):
            peer = (my + k) % N_DEV
            ag = pltpu.make_async_remote_copy(
                src_ref=ag_src,
                dst_ref=ag_buf.at[my],
                send_sem=ag_send_sems.at[k - 1],
                recv_sem=ag_recv_sems.at[my],
                device_id=(peer,), device_id_type=MESH,
            )
            ag.start()
        for k in range(1, N_DEV):
            peer = (my + k) % N_DEV
            agr = pltpu.make_async_remote_copy(
                src_ref=ag_src,
                dst_ref=ag_buf.at[peer],
                send_sem=ag_send_sems.at[0],
                recv_sem=ag_recv_sems.at[peer],
                device_id=(peer,), device_id_type=MESH,
            )
            agr.wait_recv()
            out_ref[0, pl.ds(peer * CS, CS), :] = ag_buf[peer].astype(jnp.float32)

        for s in range(N_CHUNK):
            @pl.when(my != s)
            def _(s=s):
                sd = pltpu.make_async_remote_copy(
                    src_ref=rs_send_buf.at[s],
                    dst_ref=rs_buf.at[my],
                    send_sem=rs_send_sems.at[s],
                    recv_sem=rs_recv_sems.at[my],
                    device_id=(s,), device_id_type=MESH,
                )
                sd.wait_send()
        for k in range(1, N_DEV):
            peer = (my + k) % N_DEV
            ags = pltpu.make_async_remote_copy(
                src_ref=ag_src,
                dst_ref=ag_buf.at[my],
                send_sem=ag_send_sems.at[k - 1],
                recv_sem=ag_recv_sems.at[my],
                device_id=(peer,), device_id_type=MESH,
            )
            ags.wait_send()

        for relayer, slot, dbuf, dsems in ((3, 0, k_buf, k_recv_sems),
                                           (1, 1, v_buf, v_recv_sems)):
            @pl.when(my == relayer)
            def _(slot=slot, dbuf=dbuf, dsems=dsems):
                for c in range(N_CHUNK):
                    fs = pltpu.make_async_remote_copy(
                        src_ref=relay_buf.at[c],
                        dst_ref=dbuf.at[:, pl.ds(CS * c, CS), :],
                        send_sem=relay_send_sems.at[c],
                        recv_sem=dsems.at[c],
                        device_id=(2,), device_id_type=MESH,
                    )
                    fs.wait_send()

        @pl.when(my == 0)
        def _():
            for c in range(N_CHUNK):
                for slot, (own_buf, send16, rsems) in enumerate([
                        (k_buf, ksend16, k_recv_sems),
                        (v_buf, vsend16, v_recv_sems)]):
                    for j in range(1, N_DEV):
                        r = pltpu.make_async_remote_copy(
                            src_ref=send16.at[j - 1, :, pl.ds(CS * c, CS), :],
                            dst_ref=(relay_buf.at[c] if j == 2 else
                                     own_buf.at[:, pl.ds(CS * c, CS), :]),
                            send_sem=kv_send_sems.at[c * 6 + (j - 1) * 2 + slot],
                            recv_sem=(relay_recv_sems.at[c + N_CHUNK * slot]
                                      if j == 2 else rsems.at[c]),
                            device_id=((3 - 2 * slot,) if j == 2 else (j,)),
                            device_id_type=MESH,
                        )
                        r.wait_send()

    return pl.pallas_call(
        body,
        out_shape=jax.ShapeDtypeStruct((B, Sq, D), jnp.float32),
        in_specs=[
            pl.BlockSpec(memory_space=pltpu.VMEM),
            pl.BlockSpec(memory_space=pltpu.VMEM),
            pl.BlockSpec(memory_space=pl.ANY),
            pl.BlockSpec(memory_space=pl.ANY),
            pl.BlockSpec(memory_space=pltpu.VMEM),
        ],
        out_specs=pl.BlockSpec(memory_space=pltpu.VMEM),
        scratch_shapes=[
            pltpu.VMEM((2, CS, HQ, DH), jnp.float32),
            pltpu.VMEM((N_DEV - 1, HQ_LOC, SKV, DH), jnp.bfloat16),
            pltpu.VMEM((N_DEV - 1, HQ_LOC, SKV, DH), jnp.bfloat16),
            pltpu.VMEM((HQ_LOC, SKV, DH), jnp.bfloat16),
            pltpu.VMEM((HQ_LOC, SKV, DH), jnp.bfloat16),
            pltpu.VMEM((HQ_LOC, SQ, DH), jnp.bfloat16),
            pltpu.VMEM((N_CHUNK, CS, D), jnp.bfloat16),
            pltpu.VMEM((N_DEV, CS, D), jnp.bfloat16),
            pltpu.VMEM((CS, D), jnp.bfloat16),
            pltpu.VMEM((N_DEV, CS, D), jnp.bfloat16),
            pltpu.VMEM((N_CHUNK, HQ_LOC, CS, DH), jnp.bfloat16),
            pltpu.SemaphoreType.DMA((6 * N_CHUNK,)),
            pltpu.SemaphoreType.DMA((N_CHUNK,)),
            pltpu.SemaphoreType.DMA((N_CHUNK,)),
            pltpu.SemaphoreType.DMA((N_CHUNK,)),
            pltpu.SemaphoreType.DMA((N_DEV,)),
            pltpu.SemaphoreType.DMA((N_DEV - 1,)),
            pltpu.SemaphoreType.DMA((N_DEV,)),
            pltpu.SemaphoreType.DMA((2,)),
            pltpu.SemaphoreType.DMA((2 * N_CHUNK,)),
            pltpu.SemaphoreType.DMA((N_CHUNK,)),
        ],
        compiler_params=pltpu.CompilerParams(
            collective_id=0,
            vmem_limit_bytes=100 * 1024 * 1024,
        ),
    )(x, Wq, K_ext, V_ext, Wo)


# device time: 121320 ns/iter; 1.0165x vs baseline; 1.0165x over previous
import jax
import jax.numpy as jnp
from jax import lax
from jax.experimental import pallas as pl
from jax.experimental.pallas import tpu as pltpu

N_DEV = 4
SQ = 1024
SKV = 1024
HQ = 32
HQ_LOC = 8
DH = 128
BLK = 64
N_CHUNK = 4
CS = SQ // N_CHUNK
SCALE = 0.08838834764831843
NEG = -1e9
MESH = pl.DeviceIdType.MESH


def kernel(x, Wq, K_ext, V_ext, Wo):
    B, Sq, D = x.shape

    def body(x_ref, wq_ref, k_ext_ref, v_ext_ref, wo_ref, out_ref,
             stage, ksend16, vsend16, k_buf, v_buf, q_hm,
             rs_send_buf, rs_buf, ag_src, ag_buf, relay_buf,
             kv_send_sems, k_recv_sems, v_recv_sems,
             rs_send_sems, rs_recv_sems, ag_send_sems, ag_recv_sems,
             stage_sems, relay_recv_sems, relay_send_sems):
        my = lax.axis_index("i")

        def stage_dma(ext_ref, c):
            return pltpu.make_async_copy(
                ext_ref.at[0, pl.ds(CS * c, CS)], stage.at[0],
                stage_sems.at[0])

        @pl.when(my == 0)
        def _():
            stage_dma(k_ext_ref, 0).start()

        bsem = pltpu.get_barrier_semaphore()
        for k in range(1, N_DEV):
            peer = (my + k) % N_DEV
            pl.semaphore_signal(bsem, inc=1, device_id=(peer,),
                                device_id_type=MESH)
        pl.semaphore_wait(bsem, N_DEV - 1)

        @pl.when(my == 0)
        def _():
            seq = [(c, slot) for c in range(N_CHUNK) for slot in (0, 1)]
            tensors = [(k_ext_ref, k_buf, ksend16, k_recv_sems),
                       (v_ext_ref, v_buf, vsend16, v_recv_sems)]
            for i, (c, slot) in enumerate(seq):
                    ext_ref, own_buf, send16, rsems = tensors[slot]
                    stage_dma(ext_ref, c).wait()
                    val16 = stage[0].astype(jnp.bfloat16)
                    own_buf[pl.ds(CS * c, CS)] = val16[:, 0:HQ_LOC, :]
                    for j in range(1, N_DEV):
                        send16[j - 1, pl.ds(CS * c, CS)] = (
                            val16[:, HQ_LOC * j:HQ_LOC * (j + 1), :])
                    if i + 1 < len(seq):
                        nc, nslot = seq[i + 1]
                        stage_dma(tensors[nslot][0], nc).start()
                    for j in range(1, N_DEV):
                        src = send16.at[j - 1, pl.ds(CS * c, CS)]
                        if j == 2:
                            r = pltpu.make_async_remote_copy(
                                src_ref=src,
                                dst_ref=relay_buf.at[c],
                                send_sem=kv_send_sems.at[c * 6 + 2 + slot],
                                recv_sem=relay_recv_sems.at[c + N_CHUNK * slot],
                                device_id=(3 - 2 * slot,),
                                device_id_type=MESH,
                            )
                        else:
                            r = pltpu.make_async_remote_copy(
                                src_ref=src,
                                dst_ref=own_buf.at[pl.ds(CS * c, CS)],
                                send_sem=kv_send_sems.at[c * 6 + (j - 1) * 2
                                                         + slot],
                                recv_sem=rsems.at[c],
                                device_id=(j,), device_id_type=MESH,
                            )
                        r.start()

        x16 = x_ref[0, :, :].astype(jnp.bfloat16)
        wq16 = wq_ref[:, :].astype(jnp.bfloat16)
        q2d = jnp.dot(x16, wq16, preferred_element_type=jnp.float32) * SCALE
        for h in range(HQ_LOC):
            q_hm[h] = q2d[:, h * DH:(h + 1) * DH].astype(jnp.bfloat16)
        wo16 = wo_ref[:, :].astype(jnp.bfloat16)

        keep_diag = (lax.broadcasted_iota(jnp.int32, (CS, CS), 1) // BLK
                     <= lax.broadcasted_iota(jnp.int32, (CS, CS), 0) // BLK)

        def qk_tile(s, c, h):
            qh = q_hm[h, pl.ds(CS * s, CS), :]
            kh = k_buf[CS * c:CS * (c + 1), h, :]
            sc = lax.dot_general(qh, kh, (((1,), (1,)), ((), ())),
                                 preferred_element_type=jnp.float32)
            if s == c:
                sc = jnp.where(keep_diag, sc, NEG)
            w = jnp.exp(sc)
            vh = v_buf[CS * c:CS * (c + 1), h, :]
            pv = jnp.dot(w.astype(jnp.bfloat16), vh,
                         preferred_element_type=jnp.float32)
            return pv, jnp.sum(w, axis=1, keepdims=True)

        ctx_acc = [[None] * HQ_LOC for _ in range(N_CHUNK)]
        den_acc = [[None] * HQ_LOC for _ in range(N_CHUNK)]

        def add_tile(s, c, h):
            pv, den = qk_tile(s, c, h)
            if ctx_acc[s][h] is None:
                ctx_acc[s][h], den_acc[s][h] = pv, den
            else:
                ctx_acc[s][h] = ctx_acc[s][h] + pv
                den_acc[s][h] = den_acc[s][h] + den

        for c in range(N_CHUNK):
            @pl.when(my != 0)
            def _(c=c):
                kd = pltpu.make_async_remote_copy(
                    src_ref=k_buf.at[pl.ds(CS * c, CS)],
                    dst_ref=k_buf.at[pl.ds(CS * c, CS)],
                    send_sem=kv_send_sems.at[0], recv_sem=k_recv_sems.at[c],
                    device_id=(0,), device_id_type=MESH,
                )
                kd.wait_recv()
                vd = pltpu.make_async_remote_copy(
                    src_ref=v_buf.at[pl.ds(CS * c, CS)],
                    dst_ref=v_buf.at[pl.ds(CS * c, CS)],
                    send_sem=kv_send_sems.at[1], recv_sem=v_recv_sems.at[c],
                    device_id=(0,), device_id_type=MESH,
                )
                vd.wait_recv()

            for relayer, slot, dbuf, dsems in ((3, 0, k_buf, k_recv_sems),
                                               (1, 1, v_buf, v_recv_sems)):
                @pl.when(my == relayer)
                def _(c=c, slot=slot, dbuf=dbuf, dsems=dsems):
                    rin = pltpu.make_async_remote_copy(
                        src_ref=relay_buf.at[c], dst_ref=relay_buf.at[c],
                        send_sem=kv_send_sems.at[0],
                        recv_sem=relay_recv_sems.at[c + N_CHUNK * slot],
                        device_id=(0,), device_id_type=MESH,
                    )
                    rin.wait_recv()
                    fwd = pltpu.make_async_remote_copy(
                        src_ref=relay_buf.at[c],
                        dst_ref=dbuf.at[pl.ds(CS * c, CS)],
                        send_sem=relay_send_sems.at[c],
                        recv_sem=dsems.at[c],
                        device_id=(2,), device_id_type=MESH,
                    )
                    fwd.start()

            for h in range(HQ_LOC):
                add_tile(c, c, h)
            ctx16 = jnp.concatenate(
                [ctx_acc[c][h] / den_acc[c][h] for h in range(HQ_LOC)],
                axis=1).astype(jnp.bfloat16)
            partial_c = jnp.dot(ctx16, wo16,
                                preferred_element_type=jnp.float32)

            @pl.when(my == c)
            def _(c=c, partial_c=partial_c):
                out_ref[0, CS * c:CS * (c + 1), :] = partial_c

            @pl.when(my != c)
            def _(c=c, partial_c=partial_c):
                rs_send_buf[c] = partial_c.astype(jnp.bfloat16)
                rs = pltpu.make_async_remote_copy(
                    src_ref=rs_send_buf.at[c],
                    dst_ref=rs_buf.at[my],
                    send_sem=rs_send_sems.at[c],
                    recv_sem=rs_recv_sems.at[my],
                    device_id=(c,), device_id_type=MESH,
                )
                rs.start()

            for s in range(c + 1, N_CHUNK):
                for h in range(HQ_LOC):
                    add_tile(s, c, h)

        total = out_ref[0, pl.ds(my * CS, CS), :]
        for k in range(1, N_DEV):
            peer = (my + k) % N_DEV
            rd = pltpu.make_async_remote_copy(
                src_ref=rs_send_buf.at[0],
                dst_ref=rs_buf.at[peer],
                send_sem=rs_send_sems.at[0],
                recv_sem=rs_recv_sems.at[peer],
                device_id=(peer,), device_id_type=MESH,
            )
            rd.wait_recv()
            total = total + rs_buf[peer].astype(jnp.float32)

        ag_src[...] = total.astype(jnp.bfloat16)
        out_ref[0, pl.ds(my * CS, CS), :] = total
        for k in range(1, N_DEV):
            peer = (my + k) % N_DEV
            ag = pltpu.make_async_remote_copy(
                src_ref=ag_src,
                dst_ref=ag_buf.at[my],
                send_sem=ag_send_sems.at[k - 1],
                recv_sem=ag_recv_sems.at[my],
                device_id=(peer,), device_id_type=MESH,
            )
            ag.start()
        for k in range(1, N_DEV):
            peer = (my + k) % N_DEV
            agr = pltpu.make_async_remote_copy(
                src_ref=ag_src,
                dst_ref=ag_buf.at[peer],
                send_sem=ag_send_sems.at[0],
                recv_sem=ag_recv_sems.at[peer],
                device_id=(peer,), device_id_type=MESH,
            )
            agr.wait_recv()
            out_ref[0, pl.ds(peer * CS, CS), :] = ag_buf[peer].astype(jnp.float32)

        for s in range(N_CHUNK):
            @pl.when(my != s)
            def _(s=s):
                sd = pltpu.make_async_remote_copy(
                    src_ref=rs_send_buf.at[s],
                    dst_ref=rs_buf.at[my],
                    send_sem=rs_send_sems.at[s],
                    recv_sem=rs_recv_sems.at[my],
                    device_id=(s,), device_id_type=MESH,
                )
                sd.wait_send()
        for k in range(1, N_DEV):
            peer = (my + k) % N_DEV
            ags = pltpu.make_async_remote_copy(
                src_ref=ag_src,
                dst_ref=ag_buf.at[my],
                send_sem=ag_send_sems.at[k - 1],
                recv_sem=ag_recv_sems.at[my],
                device_id=(peer,), device_id_type=MESH,
            )
            ags.wait_send()

        for relayer, slot, dbuf, dsems in ((3, 0, k_buf, k_recv_sems),
                                           (1, 1, v_buf, v_recv_sems)):
            @pl.when(my == relayer)
            def _(slot=slot, dbuf=dbuf, dsems=dsems):
                for c in range(N_CHUNK):
                    fs = pltpu.make_async_remote_copy(
                        src_ref=relay_buf.at[c],
                        dst_ref=dbuf.at[pl.ds(CS * c, CS)],
                        send_sem=relay_send_sems.at[c],
                        recv_sem=dsems.at[c],
                        device_id=(2,), device_id_type=MESH,
                    )
                    fs.wait_send()

        @pl.when(my == 0)
        def _():
            for c in range(N_CHUNK):
                for slot, (own_buf, send16, rsems) in enumerate([
                        (k_buf, ksend16, k_recv_sems),
                        (v_buf, vsend16, v_recv_sems)]):
                    for j in range(1, N_DEV):
                        r = pltpu.make_async_remote_copy(
                            src_ref=send16.at[j - 1, pl.ds(CS * c, CS)],
                            dst_ref=(relay_buf.at[c] if j == 2 else
                                     own_buf.at[pl.ds(CS * c, CS)]),
                            send_sem=kv_send_sems.at[c * 6 + (j - 1) * 2 + slot],
                            recv_sem=(relay_recv_sems.at[c + N_CHUNK * slot]
                                      if j == 2 else rsems.at[c]),
                            device_id=((3 - 2 * slot,) if j == 2 else (j,)),
                            device_id_type=MESH,
                        )
                        r.wait_send()

    return pl.pallas_call(
        body,
        out_shape=jax.ShapeDtypeStruct((B, Sq, D), jnp.float32),
        in_specs=[
            pl.BlockSpec(memory_space=pltpu.VMEM),
            pl.BlockSpec(memory_space=pltpu.VMEM),
            pl.BlockSpec(memory_space=pl.ANY),
            pl.BlockSpec(memory_space=pl.ANY),
            pl.BlockSpec(memory_space=pltpu.VMEM),
        ],
        out_specs=pl.BlockSpec(memory_space=pltpu.VMEM),
        scratch_shapes=[
            pltpu.VMEM((1, CS, HQ, DH), jnp.float32),
            pltpu.VMEM((N_DEV - 1, SKV, HQ_LOC, DH), jnp.bfloat16),
            pltpu.VMEM((N_DEV - 1, SKV, HQ_LOC, DH), jnp.bfloat16),
            pltpu.VMEM((SKV, HQ_LOC, DH), jnp.bfloat16),
            pltpu.VMEM((SKV, HQ_LOC, DH), jnp.bfloat16),
            pltpu.VMEM((HQ_LOC, SQ, DH), jnp.bfloat16),
            pltpu.VMEM((N_CHUNK, CS, D), jnp.bfloat16),
            pltpu.VMEM((N_DEV, CS, D), jnp.bfloat16),
            pltpu.VMEM((CS, D), jnp.bfloat16),
            pltpu.VMEM((N_DEV, CS, D), jnp.bfloat16),
            pltpu.VMEM((N_CHUNK, CS, HQ_LOC, DH), jnp.bfloat16),
            pltpu.SemaphoreType.DMA((6 * N_CHUNK,)),
            pltpu.SemaphoreType.DMA((N_CHUNK,)),
            pltpu.SemaphoreType.DMA((N_CHUNK,)),
            pltpu.SemaphoreType.DMA((N_CHUNK,)),
            pltpu.SemaphoreType.DMA((N_DEV,)),
            pltpu.SemaphoreType.DMA((N_DEV - 1,)),
            pltpu.SemaphoreType.DMA((N_DEV,)),
            pltpu.SemaphoreType.DMA((1,)),
            pltpu.SemaphoreType.DMA((2 * N_CHUNK,)),
            pltpu.SemaphoreType.DMA((N_CHUNK,)),
        ],
        compiler_params=pltpu.CompilerParams(
            collective_id=0,
            vmem_limit_bytes=100 * 1024 * 1024,
        ),
    )(x, Wq, K_ext, V_ext, Wo)
